# Initial kernel scaffold; baseline (speedup 1.0000x reference)
import jax
import jax.numpy as jnp
from jax.experimental import pallas as pl


def kernel(edge_index, W):
    raise NotImplementedError("write your pallas kernel here")


# speedup vs baseline: 9.1815x; 9.1815x over previous
import functools

import jax
import jax.numpy as jnp
from jax import lax
from jax.experimental import pallas as pl
from jax.experimental.pallas import tpu as pltpu
from jax.experimental.pallas import tpu_sc as plsc

NC = 2
NS = 16
NW = NC * NS
K = 128
BR = 128
D = 128
LANES = 16

def _mesh():
    return plsc.VectorSubcoreMesh(
        core_axis_name="c", subcore_axis_name="s",
        num_cores=NC, num_subcores=NS)


def _round_up(x, m):
    return (x + m - 1) // m * m


def _make_deg(nch, npad):
    rpt = npad // NS

    @functools.partial(
        pl.kernel,
        out_type=jax.ShapeDtypeStruct((NC, NS, rpt), jnp.float32),
        mesh=_mesh(),
        scratch_types=[
            pltpu.VMEM((nch, K), jnp.int32),
            pltpu.VMEM((K,), jnp.float32),
            pltpu.VMEM((rpt,), jnp.float32),
            pltpu.VMEM_SHARED((npad,), jnp.float32),
        ],
    )
    def deg_kernel(colr_hbm, out_hbm, idxc_v, ones_v, dbuf_v, dacc_sh):
        c = lax.axis_index("c")
        s = lax.axis_index("s")
        wid = c * NS + s
        zero16 = jnp.zeros((LANES,), jnp.float32)
        one16 = jnp.ones((LANES,), jnp.float32)

        def _z(i, _):
            dbuf_v[pl.ds(i * LANES, LANES)] = zero16
            return 0
        lax.fori_loop(0, rpt // LANES, _z, 0)

        def _o(i, _):
            ones_v[pl.ds(i * LANES, LANES)] = one16
            return 0
        lax.fori_loop(0, K // LANES, _o, 0)

        pltpu.sync_copy(dbuf_v, dacc_sh.at[pl.ds(s * rpt, rpt)])
        plsc.subcore_barrier()

        pltpu.sync_copy(colr_hbm.at[wid], idxc_v)

        def _chunk(j, _):
            pltpu.sync_copy(ones_v, dacc_sh.at[idxc_v.at[j]], add=True)
            return 0
        lax.fori_loop(0, nch, _chunk, 0)
        plsc.subcore_barrier()

        pltpu.sync_copy(dacc_sh.at[pl.ds(s * rpt, rpt)], dbuf_v)
        pltpu.sync_copy(dbuf_v, out_hbm.at[c, s])

    return deg_kernel


def _make_layer(nch, npad):
    rpt = npad // NS
    nblk = rpt // BR

    @functools.partial(
        pl.kernel,
        out_type=jax.ShapeDtypeStruct((NC, npad, D), jnp.float32),
        mesh=_mesh(),
        scratch_types=[
            pltpu.VMEM((nch, K), jnp.int32),
            pltpu.VMEM((nch, K), jnp.int32),
            pltpu.VMEM((K, D), jnp.float32),
            pltpu.VMEM_SHARED((npad, D), jnp.float32),
            pltpu.SemaphoreType.DMA,
        ],
    )
    def layer_kernel(z_hbm, rowr_hbm, colr_hbm, out_hbm,
                     idxr_v, idxc_v, rows_v, acc_sh, sem):
        buf_v = rows_v
        c = lax.axis_index("c")
        s = lax.axis_index("s")
        wid = c * NS + s
        zero16 = jnp.zeros((LANES,), jnp.float32)

        def _zrow(r, _):
            for j in range(D // LANES):
                buf_v[r, pl.ds(j * LANES, LANES)] = zero16
            return 0
        lax.fori_loop(0, BR, _zrow, 0)

        def _zcp(b, _):
            pltpu.sync_copy(buf_v, acc_sh.at[pl.ds(s * rpt + b * BR, BR)])
            return 0
        lax.fori_loop(0, nblk, _zcp, 0)
        plsc.subcore_barrier()

        pltpu.sync_copy(rowr_hbm.at[wid], idxr_v)
        pltpu.sync_copy(colr_hbm.at[wid], idxc_v)

        def _chunk(j, _):
            pltpu.async_copy(z_hbm.at[idxr_v.at[j]], rows_v, sem).wait()
            pltpu.sync_copy(rows_v, acc_sh.at[idxc_v.at[j]], add=True)
            return 0
        lax.fori_loop(0, nch, _chunk, 0)
        plsc.subcore_barrier()

        def _ocp(b, _):
            base = s * rpt + b * BR
            pltpu.sync_copy(acc_sh.at[pl.ds(base, BR)], buf_v)
            pltpu.sync_copy(buf_v, out_hbm.at[c, pl.ds(base, BR)])
            return 0
        lax.fori_loop(0, nblk, _ocp, 0)

    return layer_kernel


def _d_of(degp):
    deg = degp[0] + degp[1]
    return jnp.where(deg > 0.0, lax.rsqrt(deg), 0.0)


def _tc_scale0_body(degp_ref, w_ref, z0_ref):
    d = _d_of(degp_ref[...])
    z0_ref[...] = d * w_ref[...]


def _tc_mid_body(degp_ref, sp_ref, w_ref, z1_ref, acc_ref):
    d = _d_of(degp_ref[...])
    ds_ = d * (sp_ref[0] + sp_ref[1])
    z1_ref[...] = d * ds_
    acc_ref[...] = w_ref[...] + ds_


def _tc_final_body(degp_ref, sp_ref, acc_ref, out_ref):
    d = _d_of(degp_ref[...])
    out_ref[...] = (acc_ref[...] + d * (sp_ref[0] + sp_ref[1])) * (1.0 / 3.0)


def _tc_call(body, degp3, arrays, n_out, npad):
    brc = 2048
    grid = (npad // brc,)
    degp_spec = pl.BlockSpec((NC, brc, 1), lambda i: (0, i, 0))
    mat_spec = pl.BlockSpec((brc, D), lambda i: (i, 0))
    part_spec = pl.BlockSpec((NC, brc, D), lambda i: (0, i, 0))
    in_specs = [degp_spec]
    for a in arrays:
        in_specs.append(part_spec if a.ndim == 3 else mat_spec)
    out_shape = tuple(
        jax.ShapeDtypeStruct((npad, D), jnp.float32) for _ in range(n_out))
    out_specs = tuple(mat_spec for _ in range(n_out))
    if n_out == 1:
        out_shape, out_specs = out_shape[0], out_specs[0]
    return pl.pallas_call(
        body, grid=grid, in_specs=in_specs,
        out_specs=out_specs, out_shape=out_shape,
    )(degp3, *arrays)


def kernel(edge_index, W):
    n, dm = W.shape
    e = edge_index.shape[1]
    assert dm == D
    npad = _round_up(n, NS * BR)
    e_pad = _round_up(e, NW * K)
    nch = e_pad // (NW * K)

    row = edge_index[0].astype(jnp.int32)
    col = edge_index[1].astype(jnp.int32)
    pad_n = e_pad - e
    pad_idx = jnp.full((pad_n,), npad - 1, jnp.int32)
    rowr = jnp.concatenate([row, pad_idx]).reshape(NW, nch, K)
    colr = jnp.concatenate([col, pad_idx]).reshape(NW, nch, K)
    w_pad = jnp.pad(W, ((0, npad - n), (0, 0)))

    degp = _make_deg(nch, npad)(colr)
    degp3 = degp.reshape(NC, npad, 1)

    layer = _make_layer(nch, npad)
    z0 = _tc_call(_tc_scale0_body, degp3, [w_pad], 1, npad)
    s1p = layer(z0, rowr, colr)
    z1, acc01 = _tc_call(_tc_mid_body, degp3, [s1p, w_pad], 2, npad)
    s2p = layer(z1, rowr, colr)
    outp = _tc_call(_tc_final_body, degp3, [s2p, acc01], 1, npad)

    emb = outp[:n]
    nu = n // 2
    return (emb[:nu], emb[nu:])


# speedup vs baseline: 23.6979x; 2.5811x over previous
import functools

import jax
import jax.numpy as jnp
from jax import lax
from jax.experimental import pallas as pl
from jax.experimental.pallas import tpu as pltpu
from jax.experimental.pallas import tpu_sc as plsc

NC = 2
NS = 16
NW = NC * NS
K = 128
D = 128
LANES = 16


def _mesh():
    return plsc.VectorSubcoreMesh(
        core_axis_name="c", subcore_axis_name="s",
        num_cores=NC, num_subcores=NS)


def _round_up(x, m):
    return (x + m - 1) // m * m


def _make_deg(nch, npad):
    rpt = npad // NS

    @functools.partial(
        pl.kernel,
        out_type=jax.ShapeDtypeStruct((NC, NS, rpt), jnp.float32),
        mesh=_mesh(),
        scratch_types=[
            pltpu.VMEM((nch, K), jnp.int32),
            pltpu.VMEM((K,), jnp.float32),
            pltpu.VMEM((rpt,), jnp.float32),
            pltpu.VMEM_SHARED((npad,), jnp.float32),
        ],
    )
    def deg_kernel(colr_hbm, out_hbm, idxc_v, ones_v, dbuf_v, dacc_sh):
        c = lax.axis_index("c")
        s = lax.axis_index("s")
        wid = c * NS + s
        zero16 = jnp.zeros((LANES,), jnp.float32)
        one16 = jnp.ones((LANES,), jnp.float32)

        def _z(i, _):
            dbuf_v[pl.ds(i * LANES, LANES)] = zero16
            return 0
        lax.fori_loop(0, rpt // LANES, _z, 0)

        def _o(i, _):
            ones_v[pl.ds(i * LANES, LANES)] = one16
            return 0
        lax.fori_loop(0, K // LANES, _o, 0)

        pltpu.sync_copy(dbuf_v, dacc_sh.at[pl.ds(s * rpt, rpt)])
        plsc.subcore_barrier()

        pltpu.sync_copy(colr_hbm.at[wid], idxc_v)

        def _chunk(j, _):
            pltpu.sync_copy(ones_v, dacc_sh.at[idxc_v.at[j]], add=True)
            return 0
        lax.fori_loop(0, nch, _chunk, 0)
        plsc.subcore_barrier()

        pltpu.sync_copy(dacc_sh.at[pl.ds(s * rpt, rpt)], dbuf_v)
        pltpu.sync_copy(dbuf_v, out_hbm.at[c, s])

    return deg_kernel


def _make_layer(nch, npad):
    rpt = npad // NS
    nblk = rpt // K
    nch2 = nch // 2
    assert rpt % K == 0 and nch % 4 == 0

    @functools.partial(
        pl.kernel,
        out_type=jax.ShapeDtypeStruct((NC, npad, D), jnp.float32),
        mesh=_mesh(),
        scratch_types=[
            pltpu.VMEM((nch2, K), jnp.int32),
            pltpu.VMEM((nch2, K), jnp.int32),
            pltpu.VMEM((K, D), jnp.float32),
            pltpu.VMEM((K, D), jnp.float32),
            pltpu.VMEM_SHARED((npad, D), jnp.float32),
            pltpu.SemaphoreType.DMA,
            pltpu.SemaphoreType.DMA,
        ],
    )
    def layer_kernel(z_hbm, rowr_hbm, colr_hbm, out_hbm,
                     idxr_v, idxc_v, rows0_v, rows1_v, acc_sh, sem0, sem1):
        c = lax.axis_index("c")
        s = lax.axis_index("s")
        wid = c * NS + s
        zero16 = jnp.zeros((LANES,), jnp.float32)

        def _zrow(r, _):
            for j in range(D // LANES):
                rows0_v[r, pl.ds(j * LANES, LANES)] = zero16
            return 0
        lax.fori_loop(0, K, _zrow, 0)
        for b in range(nblk):
            pltpu.sync_copy(rows0_v, acc_sh.at[pl.ds(s * rpt + b * K, K)])
        plsc.subcore_barrier()

        for p in range(2):
            pltpu.sync_copy(rowr_hbm.at[wid * 2 + p], idxr_v)
            pltpu.sync_copy(colr_hbm.at[wid * 2 + p], idxc_v)

            pltpu.async_copy(z_hbm.at[idxr_v.at[0]], rows0_v, sem0)

            def _pair(jj, _):
                j0 = 2 * jj
                pltpu.async_copy(z_hbm.at[idxr_v.at[j0 + 1]], rows1_v, sem1)
                pltpu.make_async_copy(
                    z_hbm.at[idxr_v.at[j0]], rows0_v, sem0).wait()
                pltpu.sync_copy(rows0_v, acc_sh.at[idxc_v.at[j0]], add=True)

                @pl.when(j0 + 2 < nch2)
                def _():
                    pltpu.async_copy(
                        z_hbm.at[idxr_v.at[j0 + 2]], rows0_v, sem0)

                pltpu.make_async_copy(
                    z_hbm.at[idxr_v.at[j0 + 1]], rows1_v, sem1).wait()
                pltpu.sync_copy(rows1_v, acc_sh.at[idxc_v.at[j0 + 1]],
                                add=True)
                return 0
            lax.fori_loop(0, nch2 // 2, _pair, 0)
        plsc.subcore_barrier()

        for b in range(nblk):
            base = s * rpt + b * K
            pltpu.sync_copy(acc_sh.at[pl.ds(base, K)], rows0_v)
            pltpu.sync_copy(rows0_v, out_hbm.at[c, pl.ds(base, K)])

    return layer_kernel


def _d_of(degp):
    deg = degp[0] + degp[1]
    return jnp.where(deg > 0.0, lax.rsqrt(deg), 0.0)


def _tc_scale0_body(degp_ref, w_ref, z0_ref):
    d = _d_of(degp_ref[...])
    z0_ref[...] = d * w_ref[...]


def _tc_mid_body(degp_ref, sp_ref, w_ref, z1_ref, acc_ref):
    d = _d_of(degp_ref[...])
    ds_ = d * (sp_ref[0] + sp_ref[1])
    z1_ref[...] = d * ds_
    acc_ref[...] = w_ref[...] + ds_


def _tc_final_body(degp_ref, sp_ref, acc_ref, out_ref):
    d = _d_of(degp_ref[...])
    out_ref[...] = (acc_ref[...] + d * (sp_ref[0] + sp_ref[1])) * (1.0 / 3.0)


def _tc_call(body, degp3, arrays, n_out, npad):
    brc = 2048
    grid = (npad // brc,)
    degp_spec = pl.BlockSpec((NC, brc, 1), lambda i: (0, i, 0))
    mat_spec = pl.BlockSpec((brc, D), lambda i: (i, 0))
    part_spec = pl.BlockSpec((NC, brc, D), lambda i: (0, i, 0))
    in_specs = [degp_spec]
    for a in arrays:
        in_specs.append(part_spec if a.ndim == 3 else mat_spec)
    out_shape = tuple(
        jax.ShapeDtypeStruct((npad, D), jnp.float32) for _ in range(n_out))
    out_specs = tuple(mat_spec for _ in range(n_out))
    if n_out == 1:
        out_shape, out_specs = out_shape[0], out_specs[0]
    return pl.pallas_call(
        body, grid=grid, in_specs=in_specs,
        out_specs=out_specs, out_shape=out_shape,
    )(degp3, *arrays)


def kernel(edge_index, W):
    n, dm = W.shape
    e = edge_index.shape[1]
    assert dm == D
    npad = _round_up(n + 1, NS * K)
    nch = _round_up(-(-e // (NW * K)), 4)
    e_pad = NW * nch * K

    row = edge_index[0].astype(jnp.int32)
    col = edge_index[1].astype(jnp.int32)
    pad_n = e_pad - e
    spare = npad - n
    pad_idx = n + jnp.arange(pad_n, dtype=jnp.int32) % spare
    rowr = jnp.concatenate([row, pad_idx]).reshape(NW, nch, K)
    colr = jnp.concatenate([col, pad_idx]).reshape(NW, nch, K)
    w_pad = jnp.pad(W, ((0, npad - n), (0, 0)))

    degp = _make_deg(nch, npad)(colr)
    degp3 = degp.reshape(NC, npad, 1)

    rowr2 = rowr.reshape(NW * 2, nch // 2, K)
    colr2 = colr.reshape(NW * 2, nch // 2, K)

    layer = _make_layer(nch, npad)
    z0 = _tc_call(_tc_scale0_body, degp3, [w_pad], 1, npad)
    s1p = layer(z0, rowr2, colr2)
    z1, acc01 = _tc_call(_tc_mid_body, degp3, [s1p, w_pad], 2, npad)
    s2p = layer(z1, rowr2, colr2)
    outp = _tc_call(_tc_final_body, degp3, [s2p, acc01], 1, npad)

    emb = outp[:n]
    nu = n // 2
    return (emb[:nu], emb[nu:])
